# baseline (speedup 1.0000x reference)
import jax
import jax.numpy as jnp
from jax import lax
from jax.experimental import pallas as pl
from jax.experimental.pallas import tpu as pltpu
from jax.experimental.pallas import tpu_sc as plsc

_NUM_BUCKETS = 32
_N_HEADS = 16
_HALF_HEADS = 8
_Q_LEN = 2048
_K_LEN = 2048
_BQ = 1024
_W = 4096
_P_W = 4608
_QTR = _P_W // 4
_P_OFF = 2176
_THRESHOLDS = (12, 16, 23, 32, 46, 64, 91)


def _make_sc_half_kernel(h0):
    def _sc_half_kernel(table_hbm, delta_hbm, out_hbm, table_v, delta_v, p_v):
        c = lax.axis_index("c")
        s = lax.axis_index("s")
        w = s * 2 + c
        hl = w // 4
        quarter = w - 4 * hl
        pltpu.sync_copy(table_hbm, table_v)
        pltpu.sync_copy(delta_hbm, delta_v)
        delta = delta_v[...]
        head_vec = jnp.full((16,), hl + h0, jnp.int32)
        base = quarter * _QTR

        zero = jnp.zeros((16,), jnp.int32)
        one = jnp.full((16,), 1, jnp.int32)
        half_bkt = jnp.full((16,), _NUM_BUCKETS // 2, jnp.int32)
        eight = jnp.full((16,), 8, jnp.int32)

        def body(i, carry):
            start = base + i * 16 - _P_OFF
            y = lax.iota(jnp.int32, 16) + jnp.full((16,), start, jnp.int32)
            rel = y + delta
            ret = jnp.where(rel > zero, half_bkt, zero)
            rp = jnp.abs(rel)
            large = eight
            for t in _THRESHOLDS:
                large = large + jnp.where(
                    rp >= jnp.full((16,), t, jnp.int32), one, zero
                )
            bucket = ret + jnp.where(rp < eight, rp, large)
            p_v[pl.ds(i * 16, 16)] = plsc.load_gather(
                table_v, [bucket, head_vec]
            )
            return carry

        lax.fori_loop(0, _QTR // 16, body, 0)
        pltpu.sync_copy(p_v, out_hbm.at[hl, 0, pl.ds(base, _QTR)])

    return _sc_half_kernel


def _sc_lookup_half(table, delta16, h0):
    mesh = plsc.VectorSubcoreMesh(core_axis_name="c", subcore_axis_name="s")
    return pl.kernel(
        _make_sc_half_kernel(h0),
        mesh=mesh,
        out_type=jax.ShapeDtypeStruct((_HALF_HEADS, 1, _P_W), jnp.float32),
        scratch_types=[
            pltpu.VMEM((_NUM_BUCKETS, _N_HEADS), jnp.float32),
            pltpu.VMEM((16,), jnp.int32),
            pltpu.VMEM((_QTR,), jnp.float32),
        ],
        compiler_params=pltpu.CompilerParams(needs_layout_passes=False),
    )(table, delta16)


def _tc_expand_kernel(p_ref, out_ref, f8_ref, f_ref):
    i = pl.program_id(1)

    @pl.when(i == 0)
    def _build():
        for s0 in range(8):
            f8_ref[s0:s0 + 1, :] = p_ref[0, 0:1, 128 - s0:128 - s0 + _W + 128]
        for k in range(16):
            f_ref[8 * k:8 * (k + 1), :] = f8_ref[:, 128 - 8 * k:128 - 8 * k + _W]

    for u in range(_BQ // 128):
        m = 15 - (_BQ // 128) * i - u
        out_ref[0, 0, 128 * u:128 * (u + 1), :] = f_ref[
            :, pl.ds(pl.multiple_of(128 * m, 128), _K_LEN)
        ]


def _tc_expand_alias_kernel(p_ref, prev_ref, out_ref, f8_ref, f_ref):
    del prev_ref
    _tc_expand_kernel(p_ref, out_ref, f8_ref, f_ref)


_OUT_SHAPE = jax.ShapeDtypeStruct((1, _N_HEADS, _Q_LEN, _K_LEN), jnp.float32)
_SCRATCH = [
    pltpu.VMEM((8, _W + 128), jnp.float32),
    pltpu.VMEM((128, _W), jnp.float32),
]


def kernel(q_len, k_len, table):
    delta = (jnp.asarray(k_len, jnp.int32) - _K_LEN) - (
        jnp.asarray(q_len, jnp.int32) - _Q_LEN
    )
    delta16 = jnp.full((16,), delta, jnp.int32)
    p_a = _sc_lookup_half(table, delta16, 0)
    p_b = _sc_lookup_half(table, delta16, _HALF_HEADS)

    out1 = pl.pallas_call(
        _tc_expand_kernel,
        grid=(_HALF_HEADS, _Q_LEN // _BQ),
        in_specs=[pl.BlockSpec((1, 1, _P_W), lambda h, i: (h, 0, 0))],
        out_specs=pl.BlockSpec((1, 1, _BQ, _K_LEN), lambda h, i: (0, h, i, 0)),
        out_shape=_OUT_SHAPE,
        scratch_shapes=_SCRATCH,
    )(p_a)

    out = pl.pallas_call(
        _tc_expand_alias_kernel,
        grid=(_HALF_HEADS, _Q_LEN // _BQ),
        in_specs=[
            pl.BlockSpec((1, 1, _P_W), lambda h, i: (h, 0, 0)),
            pl.BlockSpec(memory_space=pl.ANY),
        ],
        out_specs=pl.BlockSpec(
            (1, 1, _BQ, _K_LEN), lambda h, i: (0, h + _HALF_HEADS, i, 0)
        ),
        out_shape=_OUT_SHAPE,
        scratch_shapes=_SCRATCH,
        input_output_aliases={1: 0},
    )(p_b, out1)
    return out


# speedup vs baseline: 1.0782x; 1.0782x over previous
import jax
import jax.numpy as jnp
from jax import lax
from jax.experimental import pallas as pl
from jax.experimental.pallas import tpu as pltpu
from jax.experimental.pallas import tpu_sc as plsc

_NUM_BUCKETS = 32
_N_HEADS = 16
_HALF_HEADS = 8
_Q_LEN = 2048
_K_LEN = 2048
_BQ = 1024
_W = 4096
_P_W = 4608
_QTR = _P_W // 4
_P_OFF = 2176
_THRESHOLDS = (12, 16, 23, 32, 46, 64, 91)


def _sc_lookup_kernel(table_hbm, delta_hbm, out_hbm, table_v, delta_v, p_v):
    c = lax.axis_index("c")
    s = lax.axis_index("s")
    w = s * 2 + c
    hl = w // 4
    quarter = w - 4 * hl
    pltpu.sync_copy(table_hbm, table_v)
    pltpu.sync_copy(delta_hbm, delta_v)
    delta = delta_v[...]
    head_vec = jnp.full((16,), hl + _HALF_HEADS, jnp.int32)
    base = quarter * _QTR

    zero = jnp.zeros((16,), jnp.int32)
    one = jnp.full((16,), 1, jnp.int32)
    half_bkt = jnp.full((16,), _NUM_BUCKETS // 2, jnp.int32)
    eight = jnp.full((16,), 8, jnp.int32)

    def body(i, carry):
        start = base + i * 16 - _P_OFF
        y = lax.iota(jnp.int32, 16) + jnp.full((16,), start, jnp.int32)
        rel = y + delta
        ret = jnp.where(rel > zero, half_bkt, zero)
        rp = jnp.abs(rel)
        large = eight
        for t in _THRESHOLDS:
            large = large + jnp.where(
                rp >= jnp.full((16,), t, jnp.int32), one, zero
            )
        bucket = ret + jnp.where(rp < eight, rp, large)
        p_v[pl.ds(i * 16, 16)] = plsc.load_gather(table_v, [bucket, head_vec])
        return carry

    lax.fori_loop(0, _QTR // 16, body, 0)
    pltpu.sync_copy(p_v, out_hbm.at[hl, 0, pl.ds(base, _QTR)])


def _sc_lookup(table, delta16):
    mesh = plsc.VectorSubcoreMesh(core_axis_name="c", subcore_axis_name="s")
    return pl.kernel(
        _sc_lookup_kernel,
        mesh=mesh,
        out_type=jax.ShapeDtypeStruct((_HALF_HEADS, 1, _P_W), jnp.float32),
        scratch_types=[
            pltpu.VMEM((_NUM_BUCKETS, _N_HEADS), jnp.float32),
            pltpu.VMEM((16,), jnp.int32),
            pltpu.VMEM((_QTR,), jnp.float32),
        ],
        compiler_params=pltpu.CompilerParams(needs_layout_passes=False),
    )(table, delta16)


def _expand_tiles(i, out_ref, f_ref):
    for u in range(_BQ // 128):
        m = 15 - (_BQ // 128) * i - u
        out_ref[0, 0, 128 * u:128 * (u + 1), :] = f_ref[
            :, pl.ds(pl.multiple_of(128 * m, 128), _K_LEN)
        ]


def _tc_inline_kernel(delta_ref, table_ref, out_ref, f8_ref, f_ref):
    i = pl.program_id(1)

    @pl.when(i == 0)
    def _build():
        delta = delta_ref[0]
        lane = jax.lax.broadcasted_iota(jnp.int32, (8, _W + 128), 1)
        sub = jax.lax.broadcasted_iota(jnp.int32, (8, _W + 128), 0)
        rel = lane - sub - 128 - (_Q_LEN - 128) + delta
        ret = jnp.where(rel > 0, _NUM_BUCKETS // 2, 0)
        rp = jnp.abs(rel)
        large = jnp.full(rel.shape, 8, jnp.int32)
        for t in _THRESHOLDS:
            large = large + (rp >= t).astype(jnp.int32)
        bucket = ret + jnp.where(rp < 8, rp, large)
        acc = jnp.zeros((8, _W + 128), jnp.float32)
        for b in range(_NUM_BUCKETS):
            acc = acc + (bucket == b).astype(jnp.float32) * table_ref[0, 0, b]
        f8_ref[:, :] = acc
        for k in range(16):
            f_ref[8 * k:8 * (k + 1), :] = f8_ref[:, 128 - 8 * k:128 - 8 * k + _W]

    _expand_tiles(i, out_ref, f_ref)


def _tc_from_p_kernel(p_ref, prev_ref, out_ref, f8_ref, f_ref):
    del prev_ref
    i = pl.program_id(1)

    @pl.when(i == 0)
    def _build():
        for s0 in range(8):
            f8_ref[s0:s0 + 1, :] = p_ref[0, 0:1, 128 - s0:128 - s0 + _W + 128]
        for k in range(16):
            f_ref[8 * k:8 * (k + 1), :] = f8_ref[:, 128 - 8 * k:128 - 8 * k + _W]

    _expand_tiles(i, out_ref, f_ref)


_OUT_SHAPE = jax.ShapeDtypeStruct((1, _N_HEADS, _Q_LEN, _K_LEN), jnp.float32)
_SCRATCH = [
    pltpu.VMEM((8, _W + 128), jnp.float32),
    pltpu.VMEM((128, _W), jnp.float32),
]


def kernel(q_len, k_len, table):
    delta = (jnp.asarray(k_len, jnp.int32) - _K_LEN) - (
        jnp.asarray(q_len, jnp.int32) - _Q_LEN
    )
    p_b = _sc_lookup(table, jnp.full((16,), delta, jnp.int32))

    table_t = jnp.reshape(jnp.transpose(table), (_N_HEADS, 1, _NUM_BUCKETS))
    grid_spec = pltpu.PrefetchScalarGridSpec(
        num_scalar_prefetch=1,
        grid=(_HALF_HEADS, _Q_LEN // _BQ),
        in_specs=[
            pl.BlockSpec((1, 1, _NUM_BUCKETS), lambda h, i, *_: (h, 0, 0)),
        ],
        out_specs=pl.BlockSpec(
            (1, 1, _BQ, _K_LEN), lambda h, i, *_: (0, h, i, 0)
        ),
        scratch_shapes=_SCRATCH,
    )
    out1 = pl.pallas_call(
        _tc_inline_kernel,
        grid_spec=grid_spec,
        out_shape=_OUT_SHAPE,
    )(jnp.reshape(delta, (1,)), table_t)

    out = pl.pallas_call(
        _tc_from_p_kernel,
        grid=(_HALF_HEADS, _Q_LEN // _BQ),
        in_specs=[
            pl.BlockSpec((1, 1, _P_W), lambda h, i: (h, 0, 0)),
            pl.BlockSpec(memory_space=pl.ANY),
        ],
        out_specs=pl.BlockSpec(
            (1, 1, _BQ, _K_LEN), lambda h, i: (0, h + _HALF_HEADS, i, 0)
        ),
        out_shape=_OUT_SHAPE,
        scratch_shapes=_SCRATCH,
        input_output_aliases={1: 0},
    )(p_b, out1)
    return out
